# baseline (speedup 1.0000x reference)
import functools
import math

import jax
import jax.numpy as jnp
from jax import lax
from jax.experimental import pallas as pl
from jax.experimental.pallas import tpu as pltpu
from jax.experimental.pallas import tpu_sc as plsc

_INFO = plsc.get_sparse_core_info()
_NC = _INFO.num_cores
_NS = _INFO.num_subcores
_NW = _NC * _NS
_L = _INFO.num_lanes

_V = 1000000
_D = 64
_SCALE = math.sqrt(_D)
_NBLK = (_V // 128)
_VTAIL = _NBLK * 128
_SROWS = _V // 2

_mesh = lambda: plsc.VectorSubcoreMesh(core_axis_name="c", subcore_axis_name="s")
_params = lambda: pltpu.CompilerParams(
    use_tc_tiling_on_sc=True, needs_layout_passes=False
)


def _wid():
    return lax.axis_index("s") * _NC + lax.axis_index("c")


def _iota16():
    return lax.iota(jnp.int32, 16)


_NCH = _NBLK // 2


@functools.partial(
    pl.kernel,
    out_type=jax.ShapeDtypeStruct((_SROWS, 128), jnp.float32),
    mesh=_mesh(),
    scratch_types=[
        [pltpu.VMEM((_D, 256), jnp.float32) for _ in range(3)],
        [pltpu.VMEM((128, 128), jnp.float32) for _ in range(3)],
        pltpu.VMEM((32, 128), jnp.float32),
        [pltpu.SemaphoreType.DMA for _ in range(3)],
        [pltpu.SemaphoreType.DMA for _ in range(3)],
    ],
    compiler_params=_params(),
)
def _pack_table(tt, tail, scratch, stg, tbuf, tailv, gsems, ssems):
    wid = _wid()
    scale = jnp.float32(_SCALE)
    base = wid * 122 + jnp.minimum(wid, 2)
    nch = 122 + jnp.where(wid < 2, 1, 0)

    row_idx = [(_iota16() + 16 * (k % 4)) for k in range(8)]

    def fire(i, bi):
        for t in range(8):
            pltpu.async_copy(
                tt.at[pl.ds(8 * t, 8), pl.ds((base + i) * 256, 256)],
                stg[bi].at[pl.ds(8 * t, 8), pl.ds(0, 256)],
                gsems[bi],
            )

    def drain(bi):
        pltpu.make_async_copy(
            tt.at[pl.ds(0, _D), pl.ds(0, 256)], stg[bi], gsems[bi]
        ).wait()

    def drain_store(bi):
        pltpu.make_async_copy(
            tbuf[bi], scratch.at[pl.ds(0, 128), pl.ds(0, 128)], ssems[bi]
        ).wait()

    fire(0, 0)
    fire(1, 1)

    @pl.loop(0, nch, step=3)
    def _blk(i0):
        for sub3 in range(3):
            i = i0 + sub3
            @pl.when(i < nch)
            def _do():
                drain(sub3)

                @pl.when(i + 2 < nch)
                def _pre():
                    fire(i + 2, (sub3 + 2) % 3)

                @pl.when(i >= 3)
                def _free():
                    drain_store(sub3)

                @plsc.parallel_loop(0, 128, unroll=8)
                def _tr(s):
                    col0 = jnp.broadcast_to((2 * s).astype(jnp.int32), (16,))
                    col1 = col0 + 1
                    for k in range(8):
                        vals = plsc.load_gather(
                            stg[sub3], [row_idx[k], col0 if k < 4 else col1]
                        )
                        tbuf[sub3][s, pl.ds(16 * k, 16)] = vals * scale

                pltpu.async_copy(
                    tbuf[sub3],
                    scratch.at[pl.ds((base + i) * 128, 128), pl.ds(0, 128)],
                    ssems[sub3],
                )

    for bi in range(3):
        drain_store(bi)

    @pl.when(wid == _NW - 1)
    def _tail():
        pltpu.sync_copy(tail, tailv)
        pltpu.sync_copy(tailv, scratch.at[pl.ds(_VTAIL // 2, 32), pl.ds(0, 128)])


@functools.partial(
    pl.kernel,
    out_type=jax.ShapeDtypeStruct((50, _D, 16384), jnp.float32),
    mesh=_mesh(),
    scratch_types=[
        pltpu.VMEM((8, 128), jnp.int32),
        pltpu.VMEM((8, 128), jnp.int32),
        [pltpu.VMEM((128, 128), jnp.float32) for _ in range(2)],
        [pltpu.VMEM((_D, 128), jnp.float32) for _ in range(2)],
        pltpu.SemaphoreType.DMA,
        [pltpu.SemaphoreType.DMA for _ in range(2)],
    ],
    compiler_params=_params(),
)
def _emb(xt, scratch, out, idx_v, sidx, rowb, tbuf, gsem, ssems):
    wid = _wid()
    i16 = _iota16()
    row_idx = [(i16 + 16 * k) for k in range(8)]

    def fire(hh, bi):
        pltpu.async_copy(scratch.at[sidx.at[hh]], rowb[bi], gsem)

    def drain(bi):
        pltpu.make_async_copy(
            scratch.at[pl.ds(0, 128), pl.ds(0, 128)], rowb[bi], gsem
        ).wait()

    def drain_store(bi):
        pltpu.make_async_copy(
            tbuf[bi],
            out.at[0, pl.ds(0, _D), pl.ds(0, 128)],
            ssems[bi],
        ).wait()

    @pl.loop(0, 28, init_carry=jnp.int32(0))
    def _unit(u, nstores):
        h8 = u >> 2
        bb = wid * 4 + (u & 3)
        hmax = jnp.minimum(8, 50 - 8 * h8)

        pltpu.sync_copy(
            xt.at[pl.ds(h8 * 8, 8), pl.ds(bb * 128, 128)], idx_v
        )
        for r in range(8):
            for k in range(8):
                sl = pl.ds(16 * k, 16)
                sidx[r, sl] = lax.shift_right_logical(idx_v[r, sl], 1)

        fire(0, 0)

        @pl.loop(0, hmax, step=2, init_carry=nstores)
        def _h(h0, ns):
            for sub in range(2):
                hh = h0 + sub
                drain(sub)

                @pl.when(hh + 1 < hmax)
                def _pre():
                    fire(hh + 1, 1 - sub)

                ns = ns + 1

                @pl.when(ns > 2)
                def _free():
                    drain_store(sub)

                pk = [
                    lax.shift_left(
                        lax.bitwise_and(idx_v[hh, pl.ds(16 * k, 16)], 1), 6
                    )
                    for k in range(8)
                ]

                @plsc.parallel_loop(0, _D, unroll=8)
                def _tr(f):
                    fb = jnp.broadcast_to(f.astype(jnp.int32), (16,))
                    for k in range(8):
                        vals = plsc.load_gather(rowb[sub], [row_idx[k], pk[k] + fb])
                        tbuf[sub][f, pl.ds(16 * k, 16)] = vals

                pltpu.async_copy(
                    tbuf[sub],
                    out.at[h8 * 8 + hh, pl.ds(0, _D), pl.ds(bb * 128, 128)],
                    ssems[sub],
                )
            return ns

        return nstores + hmax

    drain_store(0)
    drain_store(1)


@jax.jit
def _run(x, table):
    tt = table.T
    tail = table[_VTAIL:, :].reshape(32, 128) * jnp.float32(_SCALE)
    xt = jnp.pad(x.astype(jnp.int32).T, ((0, 6), (0, 0)))
    scratch = _pack_table(tt, tail)
    out = _emb(xt, scratch)
    return out.transpose(2, 0, 1)


def kernel(x, table):
    assert x.shape == (16384, 50) and table.shape == (_V, _D)
    return _run(x, table)


# speedup vs baseline: 1.3175x; 1.3175x over previous
import functools
import math

import jax
import jax.numpy as jnp
from jax import lax
from jax.experimental import pallas as pl
from jax.experimental.pallas import tpu as pltpu
from jax.experimental.pallas import tpu_sc as plsc

_INFO = plsc.get_sparse_core_info()
_NC = _INFO.num_cores
_NS = _INFO.num_subcores
_NW = _NC * _NS
_L = _INFO.num_lanes

_G = 128
_GPC = 2
_CHUNK = _G * _GPC
_NBUF = 5


@functools.partial(jax.jit, static_argnames=("n_chunks",))
def _run(idx2d, table, n_chunks):
    d = table.shape[1]
    b = idx2d.shape[0] * _G
    irows_pw = n_chunks * _GPC

    @functools.partial(
        pl.kernel,
        out_type=jax.ShapeDtypeStruct((b, d), jnp.float32),
        mesh=plsc.VectorSubcoreMesh(core_axis_name="c", subcore_axis_name="s"),
        scratch_types=[
            pltpu.VMEM((irows_pw, _G), jnp.int32),
            [pltpu.VMEM((_CHUNK, d), jnp.float32) for _ in range(_NBUF)],
            [pltpu.SemaphoreType.DMA for _ in range(_NBUF)],
            [pltpu.SemaphoreType.DMA for _ in range(_NBUF)],
        ],
        compiler_params=pltpu.CompilerParams(use_tc_tiling_on_sc=False),
    )
    def emb(idx_hbm, table_hbm, out_hbm, idx_v, rows, gsems, ssems):
        wid = lax.axis_index("s") * _NC + lax.axis_index("c")
        scale = jnp.float32(math.sqrt(d))
        pltpu.sync_copy(idx_hbm.at[pl.ds(wid * irows_pw, irows_pw)], idx_v)

        def fire_gathers(cc, bi):
            for j in range(_GPC):
                pltpu.async_copy(
                    table_hbm.at[idx_v.at[cc * _GPC + j]],
                    rows[bi].at[pl.ds(j * _G, _G)],
                    gsems[bi],
                )

        def drain_gathers(bi):
            pltpu.make_async_copy(
                table_hbm.at[pl.ds(0, _CHUNK)], rows[bi], gsems[bi]
            ).wait()

        def drain_scatter(bi):
            pltpu.make_async_copy(
                rows[bi], out_hbm.at[pl.ds(0, _CHUNK)], ssems[bi]
            ).wait()

        fire_gathers(0, 0)
        fire_gathers(1, 1)
        fire_gathers(2, 2)

        @pl.loop(0, n_chunks, step=_NBUF)
        def _step(c):
            for bi in range(_NBUF):
                cc = c + bi
                drain_gathers(bi)

                nbi = (bi + 3) % _NBUF

                @pl.when(cc + 3 < n_chunks)
                def _prefetch():
                    @pl.when(cc >= 2)
                    def _free():
                        drain_scatter(nbi)

                    fire_gathers(cc + 3, nbi)

                @plsc.parallel_loop(0, _CHUNK, unroll=8)
                def _scale(r):
                    for q in range(d // _L):
                        sl = pl.ds(q * _L, _L)
                        rows[bi][r, sl] = rows[bi][r, sl] * scale

                pltpu.async_copy(
                    rows[bi],
                    out_hbm.at[pl.ds((wid * n_chunks + cc) * _CHUNK, _CHUNK)],
                    ssems[bi],
                )

        for bi in range(_NBUF):
            drain_scatter(bi)

    return emb(idx2d, table)


def kernel(x, table):
    batch, hist = x.shape
    d = table.shape[1]
    b = batch * hist
    assert b % (_NW * _CHUNK * _NBUF) == 0 and d % _L == 0
    idx2d = x.astype(jnp.int32).reshape(b // _G, _G)
    n_chunks = b // (_NW * _CHUNK)
    out = _run(idx2d, table, n_chunks)
    return out.reshape(batch, hist, d)
